# baseline (speedup 1.0000x reference)
import jax
import jax.numpy as jnp
from jax.experimental import pallas as pl

_ITERATIONS = 2
_NEG_INF = float("-inf")


def _shift(x, d, axis):
    n = x.shape[axis]
    if axis == 0:
        pad = jnp.full((abs(d), x.shape[1]), _NEG_INF, x.dtype)
        if d > 0:
            return jnp.concatenate([pad, x[: n - d, :]], axis=0)
        return jnp.concatenate([x[-d:, :], pad], axis=0)
    pad = jnp.full((x.shape[0], abs(d)), _NEG_INF, x.dtype)
    if d > 0:
        return jnp.concatenate([pad, x[:, : n - d]], axis=1)
    return jnp.concatenate([x[:, -d:], pad], axis=1)


def _maxpool(x):
    a = jnp.maximum(x, _shift(x, 1, 0))
    a = jnp.maximum(a, _shift(a, 2, 0))
    y = jnp.maximum(a, _shift(a, -3, 0))
    b = jnp.maximum(y, _shift(y, 1, 1))
    b = jnp.maximum(b, _shift(b, 2, 1))
    z = jnp.maximum(b, _shift(b, -3, 1))
    k = y.shape[1] - (128 if y.dtype == jnp.float32 else 256)
    ye = y[:, k:]
    t = jnp.maximum(ye, _shift(ye, -1, 1))
    t = jnp.maximum(t, _shift(t, -2, 1))
    return jnp.concatenate([z[:, :k], jnp.maximum(z[:, k:], t)], axis=1)


def _nms_kernel(s_ref, o_ref):
    for i in range(s_ref.shape[0]):
        _nms_one(s_ref, o_ref, i)


def _nms_one(s_ref, o_ref, i):
    h = s_ref.shape[2]
    x = s_ref[i, 0]
    pad_f32 = jnp.full((3, x.shape[1]), _NEG_INF, x.dtype)
    xp = jnp.concatenate([x, pad_f32], axis=0)
    max_mask = xp == _maxpool(xp)
    for _ in range(_ITERATIONS):
        supp_mask = _maxpool(max_mask.astype(jnp.bfloat16)) > 0
        supp_scores = jnp.concatenate(
            [jnp.where(supp_mask[:h], 0.0, x), pad_f32], axis=0
        )
        new_max = (supp_scores == _maxpool(supp_scores)) & (~supp_mask)
        max_mask = max_mask | new_max
    o_ref[i, 0] = jnp.where(max_mask[:h], x, 0.0)


def kernel(scores):
    b, c, h, w = scores.shape
    return pl.pallas_call(
        _nms_kernel,
        grid=(b * c // 2,),
        in_specs=[pl.BlockSpec((2, 1, h, w), lambda i: (i, 0, 0, 0))],
        out_specs=pl.BlockSpec((2, 1, h, w), lambda i: (i, 0, 0, 0)),
        out_shape=jax.ShapeDtypeStruct(scores.shape, scores.dtype),
    )(scores)


# speedup vs baseline: 1.0131x; 1.0131x over previous
import jax
import jax.numpy as jnp
from jax.experimental import pallas as pl

_ITERATIONS = 2
_NEG_INF = float("-inf")


def _shift(x, d, axis):
    n = x.shape[axis]
    if axis == 0:
        pad = jnp.full((abs(d), x.shape[1]), _NEG_INF, x.dtype)
        if d > 0:
            return jnp.concatenate([pad, x[: n - d, :]], axis=0)
        return jnp.concatenate([x[-d:, :], pad], axis=0)
    pad = jnp.full((x.shape[0], abs(d)), _NEG_INF, x.dtype)
    if d > 0:
        return jnp.concatenate([pad, x[:, : n - d]], axis=1)
    return jnp.concatenate([x[:, -d:], pad], axis=1)


def _maxpool(x):
    a = jnp.maximum(x, _shift(x, 1, 0))
    a = jnp.maximum(a, _shift(a, 2, 0))
    y = jnp.maximum(a, _shift(a, -3, 0))
    b = jnp.maximum(y, _shift(y, 1, 1))
    b = jnp.maximum(b, _shift(b, 2, 1))
    z = jnp.maximum(b, _shift(b, -3, 1))
    k = y.shape[1] - (128 if y.dtype == jnp.float32 else 256)
    ye = y[:, k:]
    t = jnp.maximum(ye, _shift(ye, -1, 1))
    t = jnp.maximum(t, _shift(t, -2, 1))
    return jnp.concatenate([z[:, :k], jnp.maximum(z[:, k:], t)], axis=1)


def _nms_kernel(s_ref, o_ref):
    h = s_ref.shape[2]
    x = s_ref[0, 0]
    pad_f32 = jnp.full((3, x.shape[1]), _NEG_INF, x.dtype)
    xp = jnp.concatenate([x, pad_f32], axis=0)
    max_mask = xp == _maxpool(xp)
    for _ in range(_ITERATIONS):
        supp_mask = _maxpool(max_mask.astype(jnp.bfloat16)) > 0
        supp_scores = jnp.concatenate(
            [jnp.where(supp_mask[:h], 0.0, x), pad_f32], axis=0
        )
        new_max = (supp_scores == _maxpool(supp_scores)) & (~supp_mask)
        max_mask = max_mask | new_max
    o_ref[0, 0] = jnp.where(max_mask[:h], x, 0.0)


def kernel(scores):
    b, c, h, w = scores.shape
    return pl.pallas_call(
        _nms_kernel,
        grid=(b * c,),
        in_specs=[pl.BlockSpec((1, 1, h, w), lambda i: (i, 0, 0, 0))],
        out_specs=pl.BlockSpec((1, 1, h, w), lambda i: (i, 0, 0, 0)),
        out_shape=jax.ShapeDtypeStruct(scores.shape, scores.dtype),
    )(scores)
